# baseline (speedup 1.0000x reference)
import jax
import jax.numpy as jnp
from jax import lax
from jax.experimental import pallas as pl
from jax.experimental.pallas import tpu as pltpu
from jax.experimental.pallas import tpu_sc as plsc

D = 128
NC = 2
NS = 16
NW = NC * NS
ROWS = 5000
BLOCK = 160
ROWS_PAD = BLOCK * NW


def _gather_body(x_hbm, idx_hbm, out_hbm, idx_v, bufs, *gsem):
    wid = lax.axis_index("s") * NC + lax.axis_index("c")

    pltpu.sync_copy(idx_hbm.at[wid], idx_v)

    pltpu.async_copy(x_hbm.at[idx_v.at[0]], bufs.at[0], gsem[0])

    @pl.loop(0, BLOCK, step=2)
    def _iter(j0):
        for p in range(2):
            j = j0 + p
            pltpu.make_async_copy(
                x_hbm.at[pl.ds(0, D)], bufs.at[p], gsem[p]
            ).wait()

            @pl.when(j + 1 < BLOCK)
            def _():
                pltpu.async_copy(
                    x_hbm.at[idx_v.at[j + 1]], bufs.at[1 - p], gsem[1 - p]
                )

            @pl.when(j * NW + wid < ROWS)
            def _():
                pltpu.sync_copy(bufs.at[p], out_hbm.at[pl.ds((j * NW + wid) * D, D)])


@jax.jit
def kernel(x, edge_index):
    n_edges = edge_index.shape[1]
    idx2 = jnp.transpose(edge_index).reshape(ROWS, D)
    idx2 = jnp.zeros((ROWS_PAD, D), jnp.int32).at[:ROWS].set(idx2)
    idx3 = jnp.transpose(idx2.reshape(BLOCK, NW, D), (1, 0, 2))
    mesh = plsc.VectorSubcoreMesh(
        core_axis_name="c", subcore_axis_name="s", num_cores=NC, num_subcores=NS
    )
    run = pl.kernel(
        _gather_body,
        out_type=jax.ShapeDtypeStruct((2 * n_edges, D), jnp.float32),
        mesh=mesh,
        scratch_types=[
            pltpu.VMEM((BLOCK, D), jnp.int32),
            pltpu.VMEM((2, D, D), jnp.float32),
        ] + [pltpu.SemaphoreType.DMA] * 2,
    )
    out = run(x, idx3)
    return out.reshape(n_edges, 2 * D)


# speedup vs baseline: 4.2218x; 4.2218x over previous
import jax
import jax.numpy as jnp
from jax import lax
from jax.experimental import pallas as pl
from jax.experimental.pallas import tpu as pltpu
from jax.experimental.pallas import tpu_sc as plsc

D = 128
NC = 2
NS = 16
NW = NC * NS
CHUNKS = 2500
BLOCK = -(-CHUNKS // NW)


def _gather_body(x_hbm, ei_hbm, out_hbm, ibuf, sbuf, dbuf, sem_s, sem_d):
    wid = lax.axis_index("s") * NC + lax.axis_index("c")

    @pl.loop(0, BLOCK)
    def _iter(j):
        c = j * NW + wid

        @pl.when(c < CHUNKS)
        def _():
            e0 = c * D
            pltpu.sync_copy(ei_hbm.at[:, pl.ds(e0, D)], ibuf)
            cp_s = pltpu.async_copy(x_hbm.at[ibuf.at[0]], sbuf, sem_s)
            cp_d = pltpu.async_copy(x_hbm.at[ibuf.at[1]], dbuf, sem_d)
            cp_s.wait()
            cp_d.wait()
            pltpu.sync_copy(sbuf, out_hbm.at[pl.ds(e0, D), pl.ds(0, D)])
            pltpu.sync_copy(dbuf, out_hbm.at[pl.ds(e0, D), pl.ds(D, D)])


@jax.jit
def kernel(x, edge_index):
    n_edges = edge_index.shape[1]
    mesh = plsc.VectorSubcoreMesh(
        core_axis_name="c", subcore_axis_name="s", num_cores=NC, num_subcores=NS
    )
    run = pl.kernel(
        _gather_body,
        out_type=jax.ShapeDtypeStruct((n_edges, 2 * D), jnp.float32),
        mesh=mesh,
        scratch_types=[
            pltpu.VMEM((2, D), jnp.int32),
            pltpu.VMEM((D, D), jnp.float32),
            pltpu.VMEM((D, D), jnp.float32),
            pltpu.SemaphoreType.DMA,
            pltpu.SemaphoreType.DMA,
        ],
    )
    return run(x, edge_index)


# speedup vs baseline: 5.1974x; 1.2311x over previous
import jax
import jax.numpy as jnp
from jax import lax
from jax.experimental import pallas as pl
from jax.experimental.pallas import tpu as pltpu
from jax.experimental.pallas import tpu_sc as plsc

D = 128
NC = 2
NS = 16
NW = NC * NS
CHUNKS = 2500
BLOCK = -(-CHUNKS // NW)


def _gather_body(x_hbm, ei_hbm, out_hbm, ibufs, sbufs, dbufs, *sems):
    sem_s, sem_d = sems[:2], sems[2:]
    wid = lax.axis_index("s") * NC + lax.axis_index("c")

    def fire(c, p):
        e0 = c * D
        pltpu.sync_copy(ei_hbm.at[:, pl.ds(e0, D)], ibufs.at[p])
        pltpu.async_copy(x_hbm.at[ibufs.at[p].at[0]], sbufs.at[p], sem_s[p])
        pltpu.async_copy(x_hbm.at[ibufs.at[p].at[1]], dbufs.at[p], sem_d[p])

    fire(wid, 0)

    @pl.loop(0, BLOCK + (BLOCK % 2), step=2)
    def _iter(j0):
        for p in range(2):
            j = j0 + p
            c = j * NW + wid

            @pl.when(c < CHUNKS)
            def _():
                pltpu.make_async_copy(
                    x_hbm.at[pl.ds(0, D)], sbufs.at[p], sem_s[p]
                ).wait()
                pltpu.make_async_copy(
                    x_hbm.at[pl.ds(0, D)], dbufs.at[p], sem_d[p]
                ).wait()

                @pl.when(c + NW < CHUNKS)
                def _():
                    fire(c + NW, 1 - p)

                e0 = c * D
                pltpu.sync_copy(sbufs.at[p], out_hbm.at[pl.ds(e0, D), pl.ds(0, D)])
                pltpu.sync_copy(dbufs.at[p], out_hbm.at[pl.ds(e0, D), pl.ds(D, D)])


@jax.jit
def kernel(x, edge_index):
    n_edges = edge_index.shape[1]
    mesh = plsc.VectorSubcoreMesh(
        core_axis_name="c", subcore_axis_name="s", num_cores=NC, num_subcores=NS
    )
    run = pl.kernel(
        _gather_body,
        out_type=jax.ShapeDtypeStruct((n_edges, 2 * D), jnp.float32),
        mesh=mesh,
        scratch_types=[
            pltpu.VMEM((2, 2, D), jnp.int32),
            pltpu.VMEM((2, D, D), jnp.float32),
            pltpu.VMEM((2, D, D), jnp.float32),
        ] + [pltpu.SemaphoreType.DMA] * 4,
    )
    return run(x, edge_index)


# speedup vs baseline: 5.5371x; 1.0654x over previous
import jax
import jax.numpy as jnp
from jax import lax
from jax.experimental import pallas as pl
from jax.experimental.pallas import tpu as pltpu
from jax.experimental.pallas import tpu_sc as plsc

D = 128
NC = 2
NS = 16
NW = NC * NS
CHUNKS = 2500
BLOCK = -(-CHUNKS // NW)


def _gather_body(x_hbm, ei_hbm, out_hbm, ibufs, sbufs, dbufs, *sems):
    sem_s, sem_d = sems[0:2], sems[2:4]
    sem_i, sem_w = sems[4:6], sems[6:8]
    wid = lax.axis_index("s") * NC + lax.axis_index("c")

    def fire_idx(c, p):
        pltpu.async_copy(ei_hbm.at[:, pl.ds(c * D, D)], ibufs.at[p], sem_i[p])

    def fire_gathers(p):
        pltpu.async_copy(x_hbm.at[ibufs.at[p].at[0]], sbufs.at[p], sem_s[p])
        pltpu.async_copy(x_hbm.at[ibufs.at[p].at[1]], dbufs.at[p], sem_d[p])

    fire_idx(wid, 0)
    pltpu.make_async_copy(ei_hbm.at[:, pl.ds(0, D)], ibufs.at[0], sem_i[0]).wait()
    fire_gathers(0)

    @pl.loop(0, BLOCK + (BLOCK % 2), step=2)
    def _iter(j0):
        for p in range(2):
            j = j0 + p
            c = j * NW + wid

            @pl.when(c < CHUNKS)
            def _():
                e0 = c * D

                @pl.when(c + NW < CHUNKS)
                def _():
                    fire_idx(c + NW, 1 - p)

                @pl.when(j >= 1)
                def _():
                    pltpu.make_async_copy(
                        sbufs.at[1 - p], out_hbm.at[pl.ds(0, D), pl.ds(0, D)],
                        sem_w[1 - p],
                    ).wait()
                    pltpu.make_async_copy(
                        dbufs.at[1 - p], out_hbm.at[pl.ds(0, D), pl.ds(D, D)],
                        sem_w[1 - p],
                    ).wait()

                pltpu.make_async_copy(
                    x_hbm.at[pl.ds(0, D)], sbufs.at[p], sem_s[p]
                ).wait()
                pltpu.make_async_copy(
                    x_hbm.at[pl.ds(0, D)], dbufs.at[p], sem_d[p]
                ).wait()
                pltpu.async_copy(
                    sbufs.at[p], out_hbm.at[pl.ds(e0, D), pl.ds(0, D)], sem_w[p]
                )
                pltpu.async_copy(
                    dbufs.at[p], out_hbm.at[pl.ds(e0, D), pl.ds(D, D)], sem_w[p]
                )

                @pl.when(c + NW < CHUNKS)
                def _():
                    pltpu.make_async_copy(
                        ei_hbm.at[:, pl.ds(0, D)], ibufs.at[1 - p], sem_i[1 - p]
                    ).wait()
                    fire_gathers(1 - p)

    last_j = (CHUNKS - 1 - wid) // NW
    lp = last_j % 2

    @pl.when(last_j >= 0)
    def _():
        for q in range(2):
            @pl.when(lp == q)
            def _():
                pltpu.make_async_copy(
                    sbufs.at[q], out_hbm.at[pl.ds(0, D), pl.ds(0, D)], sem_w[q]
                ).wait()
                pltpu.make_async_copy(
                    dbufs.at[q], out_hbm.at[pl.ds(0, D), pl.ds(D, D)], sem_w[q]
                ).wait()


@jax.jit
def kernel(x, edge_index):
    n_edges = edge_index.shape[1]
    mesh = plsc.VectorSubcoreMesh(
        core_axis_name="c", subcore_axis_name="s", num_cores=NC, num_subcores=NS
    )
    run = pl.kernel(
        _gather_body,
        out_type=jax.ShapeDtypeStruct((n_edges, 2 * D), jnp.float32),
        mesh=mesh,
        scratch_types=[
            pltpu.VMEM((2, 2, D), jnp.int32),
            pltpu.VMEM((2, D, D), jnp.float32),
            pltpu.VMEM((2, D, D), jnp.float32),
        ] + [pltpu.SemaphoreType.DMA] * 8,
    )
    return run(x, edge_index)


# speedup vs baseline: 5.6177x; 1.0145x over previous
import jax
import jax.numpy as jnp
from jax import lax
from jax.experimental import pallas as pl
from jax.experimental.pallas import tpu as pltpu
from jax.experimental.pallas import tpu_sc as plsc

D = 128
NC = 2
NS = 16
NW = NC * NS
CHUNKS = 2500
BLOCK = -(-CHUNKS // NW)


IRING = 4


def _gather_body(x_hbm, ei_hbm, out_hbm, ibufs, sbufs, dbufs, *sems):
    sem_s, sem_d = sems[0:2], sems[2:4]
    sem_w, sem_i = sems[4:6], sems[6:6 + IRING]
    wid = lax.axis_index("s") * NC + lax.axis_index("c")

    def fire_idx(c, k):
        pltpu.async_copy(ei_hbm.at[:, pl.ds(c * D, D)], ibufs.at[k], sem_i[k])

    def drain_idx(k):
        pltpu.make_async_copy(
            ei_hbm.at[:, pl.ds(0, D)], ibufs.at[k], sem_i[k]
        ).wait()

    def fire_gathers(k, p):
        pltpu.async_copy(x_hbm.at[ibufs.at[k].at[0]], sbufs.at[p], sem_s[p])
        pltpu.async_copy(x_hbm.at[ibufs.at[k].at[1]], dbufs.at[p], sem_d[p])

    def drain_writes(p):
        for col in (0, D):
            pltpu.make_async_copy(
                sbufs.at[p], out_hbm.at[pl.ds(0, D), pl.ds(col, D)], sem_w[p]
            ).wait()

    fire_idx(wid, 0)
    fire_idx(wid + NW, 1)
    drain_idx(0)
    fire_gathers(0, 0)

    @pl.loop(0, BLOCK + (-BLOCK % IRING), step=IRING)
    def _iter(j0):
        for q in range(IRING):
            j = j0 + q
            p = q % 2
            c = j * NW + wid

            @pl.when(c < CHUNKS)
            def _():
                e0 = c * D

                @pl.when(c + 2 * NW < CHUNKS)
                def _():
                    fire_idx(c + 2 * NW, (q + 2) % IRING)

                @pl.when(j >= 1)
                def _():
                    drain_writes(1 - p)

                @pl.when(c + NW < CHUNKS)
                def _():
                    drain_idx((q + 1) % IRING)
                    fire_gathers((q + 1) % IRING, 1 - p)

                pltpu.make_async_copy(
                    x_hbm.at[pl.ds(0, D)], sbufs.at[p], sem_s[p]
                ).wait()
                pltpu.make_async_copy(
                    x_hbm.at[pl.ds(0, D)], dbufs.at[p], sem_d[p]
                ).wait()
                pltpu.async_copy(
                    sbufs.at[p], out_hbm.at[pl.ds(e0, D), pl.ds(0, D)], sem_w[p]
                )
                pltpu.async_copy(
                    dbufs.at[p], out_hbm.at[pl.ds(e0, D), pl.ds(D, D)], sem_w[p]
                )

    last_j = (CHUNKS - 1 - wid) // NW
    lp = last_j % 2

    @pl.when(last_j >= 0)
    def _():
        for q in range(2):
            @pl.when(lp == q)
            def _():
                pltpu.make_async_copy(
                    sbufs.at[q], out_hbm.at[pl.ds(0, D), pl.ds(0, D)], sem_w[q]
                ).wait()
                pltpu.make_async_copy(
                    dbufs.at[q], out_hbm.at[pl.ds(0, D), pl.ds(D, D)], sem_w[q]
                ).wait()


@jax.jit
def kernel(x, edge_index):
    n_edges = edge_index.shape[1]
    mesh = plsc.VectorSubcoreMesh(
        core_axis_name="c", subcore_axis_name="s", num_cores=NC, num_subcores=NS
    )
    run = pl.kernel(
        _gather_body,
        out_type=jax.ShapeDtypeStruct((n_edges, 2 * D), jnp.float32),
        mesh=mesh,
        scratch_types=[
            pltpu.VMEM((IRING, 2, D), jnp.int32),
            pltpu.VMEM((2, D, D), jnp.float32),
            pltpu.VMEM((2, D, D), jnp.float32),
        ] + [pltpu.SemaphoreType.DMA] * (6 + IRING),
    )
    return run(x, edge_index)
